# baseline (speedup 1.0000x reference)
import functools

import jax
import jax.numpy as jnp
from jax import lax
from jax.experimental import pallas as pl
from jax.experimental.pallas import tpu as pltpu
from jax.experimental.pallas import tpu_sc as plsc

NUM_CONT = 13
NUM_CAT = 26
NUM_FEAT = NUM_CONT + NUM_CAT
NUM_CLUSTERS = 256
VOCAB = 100000
DIM = 64
BATCH = 4096

_INFO = plsc.get_sparse_core_info()
NC = _INFO.num_cores
NS = _INFO.num_subcores
NW = NC * NS
NB = BATCH // NW
LANES = 16
ROWV = DIM // LANES


def _w_body(r_ref, l_ref, h_ref, e_ref, o_ref):
    r = r_ref[...][:, :, None]
    lv = l_ref[...][:, None, :]
    hv = h_ref[...][:, None, :]
    o_ref[...] = r * lv + (1.0 - r) * hv + e_ref[...]


def _make_w(r, l, h, E):
    return pl.pallas_call(
        _w_body,
        out_shape=jax.ShapeDtypeStruct((NUM_CONT, NUM_CLUSTERS, DIM), jnp.float32),
    )(r, l, h, E)


_MESH = plsc.VectorSubcoreMesh(core_axis_name="c", subcore_axis_name="s")


@functools.partial(
    pl.kernel,
    out_type=jax.ShapeDtypeStruct((BATCH, NUM_FEAT, DIM), jnp.float32),
    mesh=_MESH,
    scratch_types=[
        pltpu.VMEM((NB,), jnp.int32),
        pltpu.VMEM((NB, DIM), jnp.float32),
        pltpu.VMEM((NB,), jnp.int32),
        pltpu.VMEM((NB, LANES), jnp.float32),
        pltpu.VMEM((DIM,), jnp.float32),
        pltpu.VMEM((DIM,), jnp.float32),
        pltpu.SemaphoreType.DMA,
    ],
    compiler_params=pltpu.CompilerParams(use_tc_tiling_on_sc=False),
)
def _sc_embed(tab_hbm, w_hbm, batch_t_hbm, mask_t_hbm, zo_hbm, me_hbm, pos_hbm,
              out_hbm, idx_v, rows_v, mask_v, m_v, me_v, pos_v, sem):
    wid = lax.axis_index("s") * NC + lax.axis_index("c")
    b0 = wid * NB
    pltpu.sync_copy(me_hbm, me_v)
    me_regs = [me_v[pl.ds(c * LANES, LANES)] for c in range(ROWV)]

    def task(f, _):
        is_cont = f < NUM_CONT
        off = jnp.where(is_cont, f * NUM_CLUSTERS, (f - NUM_CONT) * VOCAB)
        pltpu.sync_copy(batch_t_hbm.at[f, pl.ds(b0, NB)], idx_v)
        pltpu.sync_copy(mask_t_hbm.at[f, pl.ds(b0, NB)], mask_v)
        pltpu.sync_copy(pos_hbm.at[f], pos_v)

        def adj(i, _):
            idx_v[pl.ds(i * LANES, LANES)] = idx_v[pl.ds(i * LANES, LANES)] + off
            return 0
        lax.fori_loop(0, NB // LANES, adj, 0)

        @pl.when(is_cont)
        def _():
            pltpu.async_copy(w_hbm.at[idx_v], rows_v, sem).wait()

        @pl.when(jnp.logical_not(is_cont))
        def _():
            pltpu.async_copy(tab_hbm.at[idx_v], rows_v, sem).wait()

        pltpu.async_copy(zo_hbm.at[mask_v], m_v, sem).wait()

        pos_regs = [pos_v[pl.ds(c * LANES, LANES)] for c in range(ROWV)]

        def fixup(rr, _):
            m = m_v[rr, pl.ds(0, LANES)]
            for c in range(ROWV):
                v = rows_v[rr, pl.ds(c * LANES, LANES)]
                rows_v[rr, pl.ds(c * LANES, LANES)] = (
                    v + pos_regs[c] + m * (me_regs[c] - v))
            return 0
        lax.fori_loop(0, NB, fixup, 0)

        pltpu.sync_copy(rows_v, out_hbm.at[pl.ds(b0, NB), f])
        return 0

    lax.fori_loop(0, NUM_FEAT, task, 0)


def kernel(batch, mask, E, l, h, r, tables, MaskEmbed, PosEmbed):
    w_flat = _make_w(r, l, h, E).reshape(NUM_CONT * NUM_CLUSTERS, DIM)
    tables_flat = tables.reshape(NUM_CAT * VOCAB, DIM)
    batch_t = batch.astype(jnp.int32).T
    mask_t = mask.astype(jnp.int32).T
    zero_one = jnp.stack([jnp.zeros((LANES,), jnp.float32),
                          jnp.ones((LANES,), jnp.float32)])
    return _sc_embed(tables_flat, w_flat, batch_t, mask_t, zero_one,
                     MaskEmbed, PosEmbed)
